# baseline (speedup 1.0000x reference)
import functools

import jax
import jax.numpy as jnp
from jax import lax
from jax.experimental import pallas as pl
from jax.experimental.pallas import tpu as pltpu
from jax.experimental.pallas import tpu_sc as plsc

N_NODES = 10000
N_EDGES = 320000
NC = 2
NS = 16
NW = NC * NS
NPAD = 10112
ROWS_PER_TILE = NPAD // NS
E_TILE = N_EDGES // NW
CHUNK = 128
NBUF = 2
KCH = 8
NCHUNK = 80
NCHUNK_ALLOC = NCHUNK + NBUF
DEG_W = 8

_mesh = plsc.VectorSubcoreMesh(core_axis_name="c", subcore_axis_name="s")
_sc_params = pltpu.CompilerParams(use_tc_tiling_on_sc=False)


def _worker_id():
    return lax.axis_index("s") * NC + lax.axis_index("c")


def _deg_body(dst_hbm, ones_hbm, zero_hbm, deg_hbm, dst_v, ones_v, acc):
    cid = lax.axis_index("c")
    sid = lax.axis_index("s")
    wid = _worker_id()
    pltpu.sync_copy(dst_hbm.at[wid], dst_v)
    pltpu.sync_copy(ones_hbm, ones_v)
    r0 = sid * ROWS_PER_TILE
    pltpu.sync_copy(zero_hbm.at[pl.ds(r0, ROWS_PER_TILE)],
                    acc.at[pl.ds(r0, ROWS_PER_TILE)])
    plsc.subcore_barrier()

    def body(j, carry):
        pltpu.sync_copy(ones_v, acc.at[dst_v.at[j]], add=True)
        return carry

    lax.fori_loop(0, NCHUNK, body, 0)
    plsc.subcore_barrier()
    pltpu.sync_copy(acc.at[pl.ds(r0, ROWS_PER_TILE)],
                    deg_hbm.at[cid, pl.ds(r0, ROWS_PER_TILE)])


def _make_deg_kernel():
    return functools.partial(
        pl.kernel,
        mesh=_mesh,
        out_type=jax.ShapeDtypeStruct((NC, NPAD, DEG_W), jnp.float32),
        compiler_params=_sc_params,
        scratch_types=[
            pltpu.VMEM((NCHUNK_ALLOC, CHUNK), jnp.int32),
            pltpu.VMEM((CHUNK, DEG_W), jnp.float32),
            pltpu.VMEM_SHARED((NPAD, DEG_W), jnp.float32),
        ],
    )(_deg_body)


def _agg_body(width, src_hbm, dst_hbm, g_hbm, zero_hbm, out_hbm,
              src_v, dst_v, buf, acc, *gsems):
    cid = lax.axis_index("c")
    sid = lax.axis_index("s")
    wid = _worker_id()
    pltpu.sync_copy(src_hbm.at[wid], src_v)
    pltpu.sync_copy(dst_hbm.at[wid], dst_v)
    r0 = sid * ROWS_PER_TILE
    pltpu.sync_copy(zero_hbm.at[pl.ds(r0, ROWS_PER_TILE)],
                    acc.at[pl.ds(r0, ROWS_PER_TILE)])
    plsc.subcore_barrier()

    def body(i, carry):
        j0 = i * NBUF
        for b in range(NBUF):
            jj = j0 + b
            pltpu.async_copy(g_hbm.at[src_v.at[jj]], buf.at[b],
                             gsems[b]).wait()
            pltpu.sync_copy(buf.at[b], acc.at[dst_v.at[jj]], add=True)
        return carry

    lax.fori_loop(0, NCHUNK // NBUF, body, 0)
    plsc.subcore_barrier()
    pltpu.sync_copy(acc.at[pl.ds(r0, ROWS_PER_TILE)],
                    out_hbm.at[cid, pl.ds(r0, ROWS_PER_TILE)])


def _make_agg_kernel(width):
    return functools.partial(
        pl.kernel,
        mesh=_mesh,
        out_type=jax.ShapeDtypeStruct((NC, NPAD, width), jnp.float32),
        compiler_params=_sc_params,
        scratch_types=[
            pltpu.VMEM((NCHUNK_ALLOC, CHUNK), jnp.int32),
            pltpu.VMEM((NCHUNK_ALLOC, CHUNK), jnp.int32),
            pltpu.VMEM((NBUF, CHUNK, width), jnp.float32),
            pltpu.VMEM_SHARED((NPAD, width), jnp.float32),
        ] + [pltpu.SemaphoreType.DMA] * NBUF,
    )(functools.partial(_agg_body, width))


def _layer1_tc(x_ref, w_ref, degp_ref, g_ref, dis_ref):
    deg = degp_ref[0, :, 0:1] + degp_ref[1, :, 0:1] + 1.0
    dis = lax.rsqrt(deg)
    h = jnp.dot(x_ref[...], w_ref[...], preferred_element_type=jnp.float32)
    g_ref[...] = h * dis
    dis_ref[...] = dis


def _layer2_tc(p_ref, g1_ref, dis_ref, b1_ref, w2_ref, g2_ref):
    s = (p_ref[0] + p_ref[1] + g1_ref[...]) * dis_ref[...] + b1_ref[...]
    z = jnp.maximum(s, 0.0)
    h2 = jnp.dot(z, w2_ref[...], preferred_element_type=jnp.float32)
    g2_ref[...] = h2 * dis_ref[...]


def _finish_tc(q_ref, g2_ref, dis_ref, b2_ref, o_ref):
    o_ref[...] = (q_ref[0] + q_ref[1] + g2_ref[...]) * dis_ref[...] + b2_ref[...]


def kernel(x, edge_index, W1, b1, W2, b2):
    f32 = jnp.float32
    hidden = W1.shape[1]
    out_ch = W2.shape[1]

    ei = edge_index.astype(jnp.int32)

    def per_tile(a):
        a = a.reshape(NW, E_TILE)
        a = jnp.pad(a, ((0, 0), (0, NCHUNK_ALLOC * CHUNK - E_TILE)),
                    constant_values=N_NODES)
        return a.reshape(NW, NCHUNK_ALLOC, CHUNK)

    srcp = per_tile(ei[0])
    dstp = per_tile(ei[1])
    xp = jnp.pad(x, ((0, NPAD - N_NODES), (0, 0)))

    ones8 = jnp.ones((CHUNK, DEG_W), f32)
    z8 = jnp.zeros((NPAD, DEG_W), f32)
    zh = jnp.zeros((NPAD, hidden), f32)
    zo = jnp.zeros((NPAD, out_ch), f32)

    degp = _make_deg_kernel()(dstp, ones8, z8)

    g1, dis = pl.pallas_call(
        _layer1_tc,
        out_shape=[
            jax.ShapeDtypeStruct((NPAD, hidden), f32),
            jax.ShapeDtypeStruct((NPAD, 1), f32),
        ],
    )(xp, W1, degp)

    P = _make_agg_kernel(hidden)(srcp, dstp, g1, zh)

    g2 = pl.pallas_call(
        _layer2_tc,
        out_shape=jax.ShapeDtypeStruct((NPAD, out_ch), f32),
    )(P, g1, dis, b1[None, :], W2)

    Q = _make_agg_kernel(out_ch)(srcp, dstp, g2, zo)

    out = pl.pallas_call(
        _finish_tc,
        out_shape=jax.ShapeDtypeStruct((NPAD, out_ch), f32),
    )(Q, g2, dis, b2[None, :])

    return out[:N_NODES]


# speedup vs baseline: 1.2784x; 1.2784x over previous
import functools

import jax
import jax.numpy as jnp
from jax import lax
from jax.experimental import pallas as pl
from jax.experimental.pallas import tpu as pltpu
from jax.experimental.pallas import tpu_sc as plsc

N_NODES = 10000
N_EDGES = 320000
NC = 2
NS = 16
NW = NC * NS
NPAD = 10112
ROWS_PER_TILE = NPAD // NS
E_TILE = N_EDGES // NW
CHUNK = 128
NCHUNK = 79
DEG_W = 8

_mesh = plsc.VectorSubcoreMesh(core_axis_name="c", subcore_axis_name="s")
_sc_params = pltpu.CompilerParams(use_tc_tiling_on_sc=False)


def _worker_id():
    return lax.axis_index("s") * NC + lax.axis_index("c")


def _deg_body(dst_hbm, ones_hbm, zero_hbm, deg_hbm, dst_v, ones_v, acc):
    cid = lax.axis_index("c")
    sid = lax.axis_index("s")
    wid = _worker_id()
    pltpu.sync_copy(dst_hbm.at[wid], dst_v)
    pltpu.sync_copy(ones_hbm, ones_v)
    r0 = sid * ROWS_PER_TILE
    pltpu.sync_copy(zero_hbm.at[pl.ds(r0, ROWS_PER_TILE)],
                    acc.at[pl.ds(r0, ROWS_PER_TILE)])
    plsc.subcore_barrier()

    def body(j, carry):
        pltpu.sync_copy(ones_v, acc.at[dst_v.at[j]], add=True)
        return carry

    lax.fori_loop(0, NCHUNK, body, 0)
    plsc.subcore_barrier()
    pltpu.sync_copy(acc.at[pl.ds(r0, ROWS_PER_TILE)],
                    deg_hbm.at[cid, pl.ds(r0, ROWS_PER_TILE)])


def _make_deg_kernel():
    return functools.partial(
        pl.kernel,
        mesh=_mesh,
        out_type=jax.ShapeDtypeStruct((NC, NPAD, DEG_W), jnp.float32),
        compiler_params=_sc_params,
        scratch_types=[
            pltpu.VMEM((NCHUNK, CHUNK), jnp.int32),
            pltpu.VMEM((CHUNK, DEG_W), jnp.float32),
            pltpu.VMEM_SHARED((NPAD, DEG_W), jnp.float32),
        ],
    )(_deg_body)


def _agg_body(width, src_hbm, dst_hbm, g_hbm, zero_hbm, out_hbm,
              src_v, dst_v, buf, acc, gsem):
    cid = lax.axis_index("c")
    sid = lax.axis_index("s")
    wid = _worker_id()
    pltpu.sync_copy(src_hbm.at[wid], src_v)
    pltpu.sync_copy(dst_hbm.at[wid], dst_v)
    r0 = sid * ROWS_PER_TILE
    pltpu.sync_copy(zero_hbm.at[pl.ds(r0, ROWS_PER_TILE)],
                    acc.at[pl.ds(r0, ROWS_PER_TILE)])
    plsc.subcore_barrier()

    def body(j, carry):
        pltpu.async_copy(g_hbm.at[src_v.at[j]], buf, gsem).wait()
        pltpu.sync_copy(buf, acc.at[dst_v.at[j]], add=True)
        return carry

    lax.fori_loop(0, NCHUNK, body, 0)
    plsc.subcore_barrier()
    pltpu.sync_copy(acc.at[pl.ds(r0, ROWS_PER_TILE)],
                    out_hbm.at[cid, pl.ds(r0, ROWS_PER_TILE)])


def _make_agg_kernel(width):
    return functools.partial(
        pl.kernel,
        mesh=_mesh,
        out_type=jax.ShapeDtypeStruct((NC, NPAD, width), jnp.float32),
        compiler_params=_sc_params,
        scratch_types=[
            pltpu.VMEM((NCHUNK, CHUNK), jnp.int32),
            pltpu.VMEM((NCHUNK, CHUNK), jnp.int32),
            pltpu.VMEM((CHUNK, width), jnp.float32),
            pltpu.VMEM_SHARED((NPAD, width), jnp.float32),
            pltpu.SemaphoreType.DMA,
        ],
    )(functools.partial(_agg_body, width))


def _layer1_tc(x_ref, w_ref, degp_ref, g_ref, dis_ref):
    deg = degp_ref[0, :, 0:1] + degp_ref[1, :, 0:1] + 1.0
    dis = lax.rsqrt(deg)
    h = jnp.dot(x_ref[...], w_ref[...], preferred_element_type=jnp.float32)
    g_ref[...] = h * dis
    dis_ref[...] = dis


def _layer2_tc(p_ref, g1_ref, dis_ref, b1_ref, w2_ref, g2_ref):
    s = (p_ref[0] + p_ref[1] + g1_ref[...]) * dis_ref[...] + b1_ref[...]
    z = jnp.maximum(s, 0.0)
    h2 = jnp.dot(z, w2_ref[...], preferred_element_type=jnp.float32)
    g2_ref[...] = h2 * dis_ref[...]


def _finish_tc(q_ref, g2_ref, dis_ref, b2_ref, o_ref):
    o_ref[...] = (q_ref[0] + q_ref[1] + g2_ref[...]) * dis_ref[...] + b2_ref[...]


def kernel(x, edge_index, W1, b1, W2, b2):
    f32 = jnp.float32
    hidden = W1.shape[1]
    out_ch = W2.shape[1]

    ei = edge_index.astype(jnp.int32)

    def per_tile(a):
        a = a.reshape(NW, E_TILE)
        head = a[:, : 78 * CHUNK].reshape(NW, 78, CHUNK)
        tail = a[:, 78 * CHUNK:]
        tail = jnp.pad(tail, ((0, 0), (0, CHUNK - tail.shape[1])),
                       constant_values=N_NODES)
        return jnp.concatenate([head, tail[:, None, :]], axis=1)

    srcp = per_tile(ei[0])
    dstp = per_tile(ei[1])
    xp = jnp.pad(x, ((0, NPAD - N_NODES), (0, 0)))

    ones8 = jnp.ones((CHUNK, DEG_W), f32)
    z8 = jnp.zeros((NPAD, DEG_W), f32)
    zh = jnp.zeros((NPAD, hidden), f32)
    zo = jnp.zeros((NPAD, out_ch), f32)

    degp = _make_deg_kernel()(dstp, ones8, z8)

    g1, dis = pl.pallas_call(
        _layer1_tc,
        out_shape=[
            jax.ShapeDtypeStruct((NPAD, hidden), f32),
            jax.ShapeDtypeStruct((NPAD, 1), f32),
        ],
    )(xp, W1, degp)

    P = _make_agg_kernel(hidden)(srcp, dstp, g1, zh)

    g2 = pl.pallas_call(
        _layer2_tc,
        out_shape=jax.ShapeDtypeStruct((NPAD, out_ch), f32),
    )(P, g1, dis, b1[None, :], W2)

    Q = _make_agg_kernel(out_ch)(srcp, dstp, g2, zo)

    out = pl.pallas_call(
        _finish_tc,
        out_shape=jax.ShapeDtypeStruct((NPAD, out_ch), f32),
    )(Q, g2, dis, b2[None, :])

    return out[:N_NODES]


# speedup vs baseline: 1.8549x; 1.4510x over previous
import functools

import jax
import jax.numpy as jnp
from jax import lax
from jax.experimental import pallas as pl
from jax.experimental.pallas import tpu as pltpu
from jax.experimental.pallas import tpu_sc as plsc

N_NODES = 10000
N_EDGES = 320000
NC = 2
NS = 16
NW = NC * NS
NPAD = 10112
ROWS_PER_TILE = NPAD // NS
E_TILE = N_EDGES // NW
CHUNK = 128
NCHUNK = 79
DEG_W = 8

_mesh = plsc.VectorSubcoreMesh(core_axis_name="c", subcore_axis_name="s")
_sc_params = pltpu.CompilerParams(use_tc_tiling_on_sc=False)


def _worker_id():
    return lax.axis_index("s") * NC + lax.axis_index("c")


def _deg_body(dst_hbm, ones_hbm, zero_hbm, deg_hbm, dst_v, ones_v, acc):
    cid = lax.axis_index("c")
    sid = lax.axis_index("s")
    wid = _worker_id()
    pltpu.sync_copy(dst_hbm.at[wid], dst_v)
    pltpu.sync_copy(ones_hbm, ones_v)
    r0 = sid * ROWS_PER_TILE
    pltpu.sync_copy(zero_hbm.at[pl.ds(r0, ROWS_PER_TILE)],
                    acc.at[pl.ds(r0, ROWS_PER_TILE)])
    plsc.subcore_barrier()

    def body(j, carry):
        pltpu.sync_copy(ones_v, acc.at[dst_v.at[j]], add=True)
        return carry

    lax.fori_loop(0, NCHUNK, body, 0)
    plsc.subcore_barrier()
    pltpu.sync_copy(acc.at[pl.ds(r0, ROWS_PER_TILE)],
                    deg_hbm.at[cid, pl.ds(r0, ROWS_PER_TILE)])


def _make_deg_kernel():
    return functools.partial(
        pl.kernel,
        mesh=_mesh,
        out_type=jax.ShapeDtypeStruct((NC, NPAD, DEG_W), jnp.float32),
        compiler_params=_sc_params,
        scratch_types=[
            pltpu.VMEM((NCHUNK, CHUNK), jnp.int32),
            pltpu.VMEM((CHUNK, DEG_W), jnp.float32),
            pltpu.VMEM_SHARED((NPAD, DEG_W), jnp.float32),
        ],
    )(_deg_body)


def _agg_body(width, src_hbm, dst_hbm, g_hbm, zero_hbm, out_hbm,
              src_v, dst_v, buf, acc, gtab, gsem):
    cid = lax.axis_index("c")
    sid = lax.axis_index("s")
    wid = _worker_id()
    pltpu.sync_copy(src_hbm.at[wid], src_v)
    pltpu.sync_copy(dst_hbm.at[wid], dst_v)
    r0 = sid * ROWS_PER_TILE
    pltpu.sync_copy(zero_hbm.at[pl.ds(r0, ROWS_PER_TILE)],
                    acc.at[pl.ds(r0, ROWS_PER_TILE)])
    pltpu.sync_copy(g_hbm.at[pl.ds(r0, ROWS_PER_TILE)],
                    gtab.at[pl.ds(r0, ROWS_PER_TILE)])
    plsc.subcore_barrier()

    def body(j, carry):
        pltpu.async_copy(gtab.at[src_v.at[j]], buf, gsem).wait()
        pltpu.sync_copy(buf, acc.at[dst_v.at[j]], add=True)
        return carry

    lax.fori_loop(0, NCHUNK, body, 0)
    plsc.subcore_barrier()
    pltpu.sync_copy(acc.at[pl.ds(r0, ROWS_PER_TILE)],
                    out_hbm.at[cid, pl.ds(r0, ROWS_PER_TILE)])


def _make_agg_kernel(width):
    return functools.partial(
        pl.kernel,
        mesh=_mesh,
        out_type=jax.ShapeDtypeStruct((NC, NPAD, width), jnp.float32),
        compiler_params=_sc_params,
        scratch_types=[
            pltpu.VMEM((NCHUNK, CHUNK), jnp.int32),
            pltpu.VMEM((NCHUNK, CHUNK), jnp.int32),
            pltpu.VMEM((CHUNK, width), jnp.float32),
            pltpu.VMEM_SHARED((NPAD, width), jnp.float32),
            pltpu.VMEM_SHARED((NPAD, width), jnp.float32),
            pltpu.SemaphoreType.DMA,
        ],
    )(functools.partial(_agg_body, width))


def _layer1_tc(x_ref, w_ref, degp_ref, g_ref, dis_ref):
    deg = degp_ref[0, :, 0:1] + degp_ref[1, :, 0:1] + 1.0
    dis = lax.rsqrt(deg)
    h = jnp.dot(x_ref[...], w_ref[...], preferred_element_type=jnp.float32)
    g_ref[...] = h * dis
    dis_ref[...] = dis


def _layer2_tc(p_ref, g1_ref, dis_ref, b1_ref, w2_ref, g2_ref):
    s = (p_ref[0] + p_ref[1] + g1_ref[...]) * dis_ref[...] + b1_ref[...]
    z = jnp.maximum(s, 0.0)
    h2 = jnp.dot(z, w2_ref[...], preferred_element_type=jnp.float32)
    g2_ref[...] = h2 * dis_ref[...]


def _finish_tc(q_ref, g2_ref, dis_ref, b2_ref, o_ref):
    o_ref[...] = (q_ref[0] + q_ref[1] + g2_ref[...]) * dis_ref[...] + b2_ref[...]


def kernel(x, edge_index, W1, b1, W2, b2):
    f32 = jnp.float32
    hidden = W1.shape[1]
    out_ch = W2.shape[1]

    ei = edge_index.astype(jnp.int32)

    def per_tile(a):
        a = a.reshape(NW, E_TILE)
        head = a[:, : 78 * CHUNK].reshape(NW, 78, CHUNK)
        tail = a[:, 78 * CHUNK:]
        tail = jnp.pad(tail, ((0, 0), (0, CHUNK - tail.shape[1])),
                       constant_values=N_NODES)
        return jnp.concatenate([head, tail[:, None, :]], axis=1)

    srcp = per_tile(ei[0])
    dstp = per_tile(ei[1])
    xp = jnp.pad(x, ((0, NPAD - N_NODES), (0, 0)))

    ones8 = jnp.ones((CHUNK, DEG_W), f32)
    z8 = jnp.zeros((NPAD, DEG_W), f32)
    zh = jnp.zeros((NPAD, hidden), f32)
    zo = jnp.zeros((NPAD, out_ch), f32)

    degp = _make_deg_kernel()(dstp, ones8, z8)

    g1, dis = pl.pallas_call(
        _layer1_tc,
        out_shape=[
            jax.ShapeDtypeStruct((NPAD, hidden), f32),
            jax.ShapeDtypeStruct((NPAD, 1), f32),
        ],
    )(xp, W1, degp)

    P = _make_agg_kernel(hidden)(srcp, dstp, g1, zh)

    g2 = pl.pallas_call(
        _layer2_tc,
        out_shape=jax.ShapeDtypeStruct((NPAD, out_ch), f32),
    )(P, g1, dis, b1[None, :], W2)

    Q = _make_agg_kernel(out_ch)(srcp, dstp, g2, zo)

    out = pl.pallas_call(
        _finish_tc,
        out_shape=jax.ShapeDtypeStruct((NPAD, out_ch), f32),
    )(Q, g2, dis, b2[None, :])

    return out[:N_NODES]
